# Initial kernel scaffold; baseline (speedup 1.0000x reference)
import jax
import jax.numpy as jnp
from jax.experimental import pallas as pl


def kernel(adj_rows, adj_cols, adj_vals, user_emb, item_emb):
    raise NotImplementedError("write your pallas kernel here")


# speedup vs baseline: 13.0956x; 13.0956x over previous
import functools

import jax
import jax.numpy as jnp
from jax import lax
from jax.experimental import pallas as pl
from jax.experimental.pallas import tpu as pltpu
from jax.experimental.pallas import tpu_sc as plsc

N_USERS = 60000
N_ITEMS = 40000
N = N_USERS + N_ITEMS
E = 1600000
EMB = 32
HALF = 16
N_LAYERS = 3

NTILES = 16
BLK = 128
BPT = 784
E_PAD = NTILES * BPT * BLK
CB = 8
CHUNKS = BPT // CB
CHUNK_E = CB * BLK

RPT = N // NTILES
ZROWS = 625
FCH = 250


def _gcn_body(cols_hbm, rows_hbm, vals_hbm, emb_hbm,
              final_hbm, t0_hbm, t1_hbm, t2_hbm, t3_hbm,
              acc, cols_v, rows_v, vals_v, gbuf, sem):
    h = lax.axis_index("c")
    t = lax.axis_index("s")
    tables = [t0_hbm, t1_hbm, t2_hbm, t3_hbm]
    r0 = t * RPT

    for k in range(RPT // FCH):
        sub = r0 + k * FCH
        pltpu.sync_copy(emb_hbm.at[pl.ds(sub, FCH), pl.ds(h * HALF, HALF)],
                        gbuf.at[pl.ds(0, FCH)])
        pltpu.sync_copy(gbuf.at[pl.ds(0, FCH)],
                        t0_hbm.at[h].at[pl.ds(sub, FCH)])

    plsc.subcore_barrier()

    def layer(src_tbl, dst_tbl):
        zrow = jnp.zeros((HALF,), jnp.float32)

        @pl.loop(0, ZROWS)
        def _(r):
            gbuf[r] = zrow

        for k in range(RPT // ZROWS):
            pltpu.sync_copy(gbuf.at[pl.ds(0, ZROWS)],
                            acc.at[pl.ds(r0 + k * ZROWS, ZROWS)])
        plsc.subcore_barrier()

        src_half = src_tbl.at[h]

        @pl.loop(0, CHUNKS)
        def _(c):
            b0 = t * BPT + c * CB
            pltpu.sync_copy(cols_hbm.at[pl.ds(b0, CB)], cols_v)
            pltpu.sync_copy(rows_hbm.at[pl.ds(b0, CB)], rows_v)
            pltpu.sync_copy(vals_hbm.at[pl.ds(b0, CB)], vals_v)
            descs = [
                pltpu.async_copy(src_half.at[cols_v.at[j]],
                                 gbuf.at[pl.ds(j * BLK, BLK)], sem)
                for j in range(CB)
            ]
            for j in range(CB):
                descs[j].wait()

                @pl.loop(0, BLK // 16)
                def _(g, j=j):
                    base = j * BLK + g * 16
                    vv = vals_v[j, pl.ds(g * 16, 16)]
                    for i in range(16):
                        gbuf[base + i] = gbuf[base + i] * vv[i]

                pltpu.sync_copy(gbuf.at[pl.ds(j * BLK, BLK)],
                                acc.at[rows_v.at[j]], add=True)

        plsc.subcore_barrier()
        pltpu.sync_copy(acc.at[pl.ds(r0, RPT)], dst_tbl.at[h].at[pl.ds(r0, RPT)])
        plsc.subcore_barrier()

    layer(t0_hbm, t1_hbm)
    layer(t1_hbm, t2_hbm)
    layer(t2_hbm, t3_hbm)

    accum = gbuf.at[pl.ds(0, FCH)]
    lbuf = gbuf.at[pl.ds(FCH, FCH)]
    for k in range(RPT // FCH):
        sub = r0 + k * FCH
        pltpu.sync_copy(tables[0].at[h].at[pl.ds(sub, FCH)], accum)
        for l in range(1, N_LAYERS + 1):
            pltpu.sync_copy(tables[l].at[h].at[pl.ds(sub, FCH)], lbuf)

            @pl.loop(0, FCH)
            def _(r):
                accum[r] = accum[r] + lbuf[r]

        @pl.loop(0, FCH)
        def _(r):
            accum[r] = accum[r] * 0.25

        pltpu.sync_copy(accum,
                        final_hbm.at[pl.ds(sub, FCH), pl.ds(h * HALF, HALF)])


@jax.jit
def _gcn(cols2d, rows2d, vals2d, all_emb):
    mesh = plsc.VectorSubcoreMesh(core_axis_name="c", subcore_axis_name="s")
    f32 = jnp.float32
    out_type = (
        jax.ShapeDtypeStruct((N, EMB), f32),
        jax.ShapeDtypeStruct((2, N, HALF), f32),
        jax.ShapeDtypeStruct((2, N, HALF), f32),
        jax.ShapeDtypeStruct((2, N, HALF), f32),
        jax.ShapeDtypeStruct((2, N, HALF), f32),
    )
    scratch = [
        pltpu.VMEM_SHARED((N, HALF), f32),
        pltpu.VMEM((CB, BLK), jnp.int32),
        pltpu.VMEM((CB, BLK), jnp.int32),
        pltpu.VMEM((CB, BLK), f32),
        pltpu.VMEM((CHUNK_E, HALF), f32),
        pltpu.SemaphoreType.DMA,
    ]
    run = pl.kernel(_gcn_body, out_type=out_type, mesh=mesh,
                    scratch_types=scratch,
                    compiler_params=pltpu.CompilerParams(
                        use_tc_tiling_on_sc=False))
    return run(cols2d, rows2d, vals2d, all_emb)


def kernel(adj_rows, adj_cols, adj_vals, user_emb, item_emb):
    all_emb = jnp.concatenate([user_emb, item_emb], axis=0)
    pad = E_PAD - E
    cols2d = jnp.pad(adj_cols, (0, pad)).reshape(E_PAD // BLK, BLK)
    rows2d = jnp.pad(adj_rows, (0, pad)).reshape(E_PAD // BLK, BLK)
    vals2d = jnp.pad(adj_vals, (0, pad)).reshape(E_PAD // BLK, BLK)
    final, _, _, _, _ = _gcn(cols2d, rows2d, vals2d, all_emb)
    return (final[:N_USERS], final[N_USERS:])
